# baseline (speedup 1.0000x reference)
import functools

import jax
import jax.numpy as jnp
import numpy as np
from jax import lax
from jax.experimental import pallas as pl
from jax.experimental.pallas import tpu as pltpu
from jax.experimental.pallas import tpu_sc as plsc

_B, _H, _V = 4096, 20, 1000
_NC, _NS = 2, 16
_NW = _NC * _NS
_BPW = _B // _NW
_CH = 2
_ROWS = _CH * _H
_NCHUNK = _BPW // _CH
_NBUF = 2

def _scatter_val(buf_b, idxs, base, value):
    val = jnp.full((16,), value, jnp.float32)
    lane = lax.iota(jnp.int32, 16)
    hconst = jnp.full((16,), _H, jnp.int32)
    for t in range(3):
        cols = idxs[pl.ds(base + t * 16, 16)]
        r = lax.add(lane, jnp.full((16,), t * 16, jnp.int32))
        bvec = lax.div(r, hconst)
        hvec = lax.rem(r, hconst)
        mask = None if t < 2 else (lane < jnp.full((16,), 8, jnp.int32))
        plsc.store_scatter(buf_b, [bvec, hvec, cols], val, mask=mask)


def _sc_body(xf_hbm, z_hbm, o_hbm, idxs, buf, sems):
    w = lax.axis_index("s") * _NC + lax.axis_index("c")
    row0 = w * _BPW * _H

    pltpu.sync_copy(xf_hbm.at[pl.ds(row0, _BPW * _H)], idxs.at[pl.ds(0, _BPW * _H)])
    for b in range(_NBUF):
        pltpu.sync_copy(z_hbm, buf.at[b])

    def chunk(c, b):
        batch = w * _BPW + c * _CH
        dst = o_hbm.at[pl.ds(batch, _CH)]

        @pl.when(c >= _NBUF)
        def _():
            pltpu.make_async_copy(buf.at[b], dst, sems.at[b]).wait()
            _scatter_val(buf.at[b], idxs, (c - _NBUF) * _ROWS, 0.0)

        _scatter_val(buf.at[b], idxs, c * _ROWS, 1.0)
        pltpu.make_async_copy(buf.at[b], dst, sems.at[b]).start()

    def group(g, carry):
        for b in range(_NBUF):
            chunk(g * _NBUF + b, b)
        return carry

    lax.fori_loop(0, _NCHUNK // _NBUF, group, 0)

    for b in range(_NBUF):
        pltpu.make_async_copy(
            buf.at[b], o_hbm.at[pl.ds(w * _BPW, _CH)], sems.at[b]
        ).wait()


@functools.partial(
    pl.kernel,
    out_type=jax.ShapeDtypeStruct((_B, _H, _V), jnp.float32),
    mesh=plsc.VectorSubcoreMesh(core_axis_name="c", subcore_axis_name="s"),
    compiler_params=pltpu.CompilerParams(
        use_tc_tiling_on_sc=False, needs_layout_passes=False
    ),
    scratch_types=[
        pltpu.VMEM((_BPW * _H + 16,), jnp.int32),
        pltpu.VMEM((_NBUF, _CH, _H, _V), jnp.float32),
        pltpu.SemaphoreType.DMA((_NBUF,)),
    ],
)
def _sc_onehot(xf_hbm, z_hbm, o_hbm, idxs, buf, sems):
    _sc_body(xf_hbm, z_hbm, o_hbm, idxs, buf, sems)


def kernel(x, W):
    xf = x.reshape(-1)
    z = jnp.zeros((_CH, _H, _V), jnp.float32)
    return _sc_onehot(xf, z)


# speedup vs baseline: 1.4694x; 1.4694x over previous
import functools

import jax
import jax.numpy as jnp
import numpy as np
from jax import lax
from jax.experimental import pallas as pl
from jax.experimental.pallas import tpu as pltpu
from jax.experimental.pallas import tpu_sc as plsc

_B, _H, _V = 4096, 20, 1000
_NC, _NS = 2, 16
_NW = _NC * _NS
_BPW = _B // _NW
_CH = 2
_ROWS = _CH * _H
_NCHUNK = _BPW // _CH
_NBUF = 2

def _scatter_val(buf_b, idxs, base, value):
    val = jnp.full((16,), value, jnp.float32)
    lane = lax.iota(jnp.int32, 16)
    hconst = jnp.full((16,), _H, jnp.int32)
    for t in range(3):
        cols = idxs[pl.ds(base + t * 16, 16)]
        r = lax.add(lane, jnp.full((16,), t * 16, jnp.int32))
        bvec = lax.div(r, hconst)
        hvec = lax.rem(r, hconst)
        mask = None if t < 2 else (lane < jnp.full((16,), 8, jnp.int32))
        plsc.store_scatter(buf_b, [bvec, hvec, cols], val, mask=mask)


def _sc_body(xf_hbm, z_hbm, o_hbm, idxs, buf, sems):
    w = lax.axis_index("s") * _NC + lax.axis_index("c")
    row0 = w * _BPW * _H

    pltpu.sync_copy(xf_hbm.at[pl.ds(row0, _BPW * _H)], idxs.at[pl.ds(0, _BPW * _H)])
    for b in range(_NBUF):
        pltpu.sync_copy(z_hbm, buf.at[b])

    def chunk(c, b):
        batch = w * _BPW + c * _CH
        dst = o_hbm.at[pl.ds(batch, _CH)]

        @pl.when(c >= _NBUF)
        def _():
            pltpu.make_async_copy(buf.at[b], dst, sems.at[b]).wait()
            _scatter_val(buf.at[b], idxs, (c - _NBUF) * _ROWS, 0.0)

        _scatter_val(buf.at[b], idxs, c * _ROWS, 1.0)
        pltpu.make_async_copy(buf.at[b], dst, sems.at[b]).start()

    def group(g, carry):
        for b in range(_NBUF):
            chunk(g * _NBUF + b, b)
        return carry

    lax.fori_loop(0, _NCHUNK // _NBUF, group, 0)

    for b in range(_NBUF):
        pltpu.make_async_copy(
            buf.at[b], o_hbm.at[pl.ds(w * _BPW, _CH)], sems.at[b]
        ).wait()


@functools.partial(
    pl.kernel,
    out_type=jax.ShapeDtypeStruct((_B, _H, _V), jnp.float32),
    mesh=plsc.VectorSubcoreMesh(core_axis_name="c", subcore_axis_name="s"),
    compiler_params=pltpu.CompilerParams(
        use_tc_tiling_on_sc=True, needs_layout_passes=False
    ),
    scratch_types=[
        pltpu.VMEM((_BPW * _H + 16,), jnp.int32),
        pltpu.VMEM((_NBUF, _CH, _H, _V), jnp.float32),
        pltpu.SemaphoreType.DMA((_NBUF,)),
    ],
)
def _sc_onehot(xf_hbm, z_hbm, o_hbm, idxs, buf, sems):
    _sc_body(xf_hbm, z_hbm, o_hbm, idxs, buf, sems)


def kernel(x, W):
    xf = x.reshape(-1)
    z = jnp.zeros((_CH, _H, _V), jnp.float32)
    return _sc_onehot(xf, z)


# speedup vs baseline: 4.8315x; 3.2881x over previous
import functools

import jax
import jax.numpy as jnp
from jax import lax
from jax.experimental import pallas as pl
from jax.experimental.pallas import tpu as pltpu
from jax.experimental.pallas import tpu_sc as plsc

_B, _H, _V = 4096, 20, 1000
_NC, _NS = 2, 16
_NW = _NC * _NS
_BT = _B // _NW


def _scatter_strip(buf, idx_ref, value):
    val = jnp.full((16,), value, jnp.float32)
    lane = lax.iota(jnp.int32, 16)
    for k in range(_BT // 16):
        cv = idx_ref[pl.ds(k * 16, 16)]
        blocal = lax.add(lane, jnp.full((16,), k * 16, jnp.int32))
        plsc.store_scatter(buf, [cv, blocal], val)


def _sc_body(xt_hbm, z_hbm, o_hbm, idxs, buf, sem):
    w = lax.axis_index("s") * _NC + lax.axis_index("c")
    b0 = w * _BT

    pltpu.sync_copy(z_hbm, buf)

    for h in range(_H):
        pltpu.sync_copy(xt_hbm.at[pl.ds(h * _B + b0, _BT)], idxs)
        _scatter_strip(buf, idxs, 1.0)
        pltpu.make_async_copy(
            buf, o_hbm.at[h, :, pl.ds(b0, _BT)], sem
        ).start()
        pltpu.make_async_copy(
            buf, o_hbm.at[h, :, pl.ds(b0, _BT)], sem
        ).wait()
        _scatter_strip(buf, idxs, 0.0)


@functools.partial(
    pl.kernel,
    out_type=jax.ShapeDtypeStruct((_H, _V, _B), jnp.float32),
    mesh=plsc.VectorSubcoreMesh(core_axis_name="c", subcore_axis_name="s"),
    compiler_params=pltpu.CompilerParams(
        use_tc_tiling_on_sc=True, needs_layout_passes=False
    ),
    scratch_types=[
        pltpu.VMEM((_BT,), jnp.int32),
        pltpu.VMEM((_V, _BT), jnp.float32),
        pltpu.SemaphoreType.DMA,
    ],
)
def _sc_onehot(xt_hbm, z_hbm, o_hbm, idxs, buf, sem):
    _sc_body(xt_hbm, z_hbm, o_hbm, idxs, buf, sem)


def kernel(x, W):
    xt = x.T.reshape(-1)
    z = jnp.zeros((_V, _BT), jnp.float32)
    out = _sc_onehot(xt, z)
    return jnp.transpose(out, (2, 0, 1))


# speedup vs baseline: 4.9568x; 1.0259x over previous
import functools

import jax
import jax.numpy as jnp
from jax import lax
from jax.experimental import pallas as pl
from jax.experimental.pallas import tpu as pltpu
from jax.experimental.pallas import tpu_sc as plsc

_B, _H, _V = 4096, 20, 1000
_NC, _NS = 2, 16
_NW = _NC * _NS
_BT = _B // _NW
_V0 = 504
_HALVES = ((0, _V0), (_V0, _V - _V0))


def _scatter_half(buf_h, idx_ref, h, lo, sz, value):
    val = jnp.full((16,), value, jnp.float32)
    lane = lax.iota(jnp.int32, 16)
    lane20 = lax.mul(lane, jnp.full((16,), _H, jnp.int32))
    lov = jnp.full((16,), lo, jnp.int32)
    hiv = jnp.full((16,), lo + sz, jnp.int32)
    for k in range(_BT // 16):
        gidx = lax.add(lane20, jnp.full((16,), (16 * k) * _H + h, jnp.int32))
        cv = plsc.load_gather(idx_ref, [gidx])
        m = jnp.logical_and(cv >= lov, cv < hiv)
        row = lax.sub(cv, lov)
        blocal = lax.add(lane, jnp.full((16,), k * 16, jnp.int32))
        plsc.store_scatter(buf_h, [row, blocal], val, mask=m)


def _sc_body(xf_hbm, z_hbm, o_hbm, idxs, buf0, buf1, sems):
    w = lax.axis_index("s") * _NC + lax.axis_index("c")
    b0 = w * _BT
    bufs = (buf0, buf1)

    pltpu.sync_copy(xf_hbm.at[pl.ds(b0 * _H, _BT * _H)], idxs)
    for i, (lo, sz) in enumerate(_HALVES):
        pltpu.sync_copy(z_hbm.at[pl.ds(0, sz)], bufs[i])

    for step in range(2 * _H + 2):
        i = step % 2
        lo, sz = _HALVES[i]
        buf = bufs[i]
        dst = lambda h: o_hbm.at[h, pl.ds(lo, sz), pl.ds(b0, _BT)]
        if step >= 2:
            h_prev = (step - 2) // 2
            pltpu.make_async_copy(buf, dst(h_prev), sems.at[i]).wait()
            _scatter_half(buf, idxs, h_prev, lo, sz, 0.0)
        if step < 2 * _H:
            h = step // 2
            _scatter_half(buf, idxs, h, lo, sz, 1.0)
            pltpu.make_async_copy(buf, dst(h), sems.at[i]).start()


@functools.partial(
    pl.kernel,
    out_type=jax.ShapeDtypeStruct((_H, _V, _B), jnp.float32),
    mesh=plsc.VectorSubcoreMesh(core_axis_name="c", subcore_axis_name="s"),
    compiler_params=pltpu.CompilerParams(
        use_tc_tiling_on_sc=True, needs_layout_passes=False
    ),
    scratch_types=[
        pltpu.VMEM((_BT * _H,), jnp.int32),
        pltpu.VMEM((_V0, _BT), jnp.float32),
        pltpu.VMEM((_V - _V0, _BT), jnp.float32),
        pltpu.SemaphoreType.DMA((2,)),
    ],
)
def _sc_onehot(xf_hbm, z_hbm, o_hbm, idxs, buf0, buf1, sems):
    _sc_body(xf_hbm, z_hbm, o_hbm, idxs, buf0, buf1, sems)


def kernel(x, W):
    xf = x.reshape(-1)
    z = jnp.zeros((_V0, _BT), jnp.float32)
    out = _sc_onehot(xf, z)
    return jnp.transpose(out, (2, 0, 1))
